# baseline (speedup 1.0000x reference)
import functools

import jax
import jax.numpy as jnp
from jax import lax
from jax.experimental import pallas as pl
from jax.experimental.pallas import tpu as pltpu
from jax.experimental.pallas import tpu_sc as plsc

N = 10000
E = 160000
D = 256
H = 128
NP = 10240
EB = 128
EPAD = 163840
ROWS = EPAD // EB
RPT = ROWS // 16
RPT_HALF = ROWS // 32
STRIPE = NP // 16

_mesh = plsc.VectorSubcoreMesh(core_axis_name="c", subcore_axis_name="s")


@functools.partial(
    pl.kernel,
    out_type=(
        jax.ShapeDtypeStruct((NP,), jnp.float32),
        jax.ShapeDtypeStruct((NP,), jnp.float32),
    ),
    mesh=_mesh,
    scratch_types=[
        pltpu.VMEM((RPT_HALF, EB), jnp.int32),
        pltpu.VMEM((RPT_HALF, EB), jnp.float32),
        pltpu.VMEM((STRIPE,), jnp.float32),
        pltpu.VMEM_SHARED((NP,), jnp.float32),
        pltpu.SemaphoreType.DMA,
    ],
)
def _deg_kernel(dst_hbm, ew_hbm, d0_hbm, d1_hbm, idx_v, val_v, stripe_v, deg_sh,
                dsem):
    cid = lax.axis_index("c")
    sid = lax.axis_index("s")
    for i in range(STRIPE // 16):
        stripe_v[pl.ds(i * 16, 16)] = jnp.zeros((16,), jnp.float32)
    pltpu.sync_copy(stripe_v, deg_sh.at[pl.ds(sid * STRIPE, STRIPE)])
    plsc.subcore_barrier()
    row0 = cid * (ROWS // 2) + sid * RPT_HALF
    pltpu.sync_copy(dst_hbm.at[pl.ds(row0, RPT_HALF)], idx_v)
    pltpu.sync_copy(ew_hbm.at[pl.ds(row0, RPT_HALF)], val_v)
    for j in range(RPT_HALF):
        pltpu.async_copy(val_v.at[j], deg_sh.at[idx_v.at[j]], dsem, add=True)
    for j in range(RPT_HALF):
        pltpu.make_async_copy(val_v.at[j], deg_sh.at[idx_v.at[j]], dsem).wait()
    plsc.subcore_barrier()
    pltpu.sync_copy(deg_sh.at[pl.ds(sid * STRIPE, STRIPE)], stripe_v)

    @pl.when(cid == 0)
    def _():
        pltpu.sync_copy(stripe_v, d0_hbm.at[pl.ds(sid * STRIPE, STRIPE)])

    @pl.when(cid == 1)
    def _():
        pltpu.sync_copy(stripe_v, d1_hbm.at[pl.ds(sid * STRIPE, STRIPE)])


def _mm_body(node_ref, w_ref, d0_ref, d1_ref, h2_ref, dinv_ref):
    deg = d0_ref[...] + d1_ref[...] + 1.0
    dinv = lax.rsqrt(deg)
    h = jnp.dot(node_ref[...], w_ref[...], preferred_element_type=jnp.float32)
    h2 = h * dinv
    h2_ref[...] = jnp.stack([h2[:, :H], h2[:, H:]])
    dinv_ref[...] = dinv


def _matmul_prescale(node, W, d0, d1, block=200):
    grid = (N // block,)
    return pl.pallas_call(
        _mm_body,
        grid=grid,
        in_specs=[
            pl.BlockSpec((block, D), lambda i: (i, 0)),
            pl.BlockSpec((D, D), lambda i: (0, 0)),
            pl.BlockSpec((block, 1), lambda i: (i, 0)),
            pl.BlockSpec((block, 1), lambda i: (i, 0)),
        ],
        out_specs=[
            pl.BlockSpec((2, block, H), lambda i: (0, i, 0)),
            pl.BlockSpec((block, 1), lambda i: (i, 0)),
        ],
        out_shape=[
            jax.ShapeDtypeStruct((2, N, H), jnp.float32),
            jax.ShapeDtypeStruct((N, 1), jnp.float32),
        ],
    )(node, W, d0, d1)


RPP = RPT // 2


@functools.partial(
    pl.kernel,
    out_type=jax.ShapeDtypeStruct((2, NP, H), jnp.float32),
    mesh=_mesh,
    scratch_types=[
        pltpu.VMEM((RPP, EB), jnp.int32),
        pltpu.VMEM((RPP, EB), jnp.int32),
        pltpu.VMEM((RPP, EB), jnp.float32),
        pltpu.VMEM((EB, H), jnp.float32),
        pltpu.VMEM((EB, H), jnp.float32),
        pltpu.VMEM_SHARED((NP, H), jnp.float32),
        pltpu.SemaphoreType.DMA,
        pltpu.SemaphoreType.DMA,
        pltpu.SemaphoreType.DMA,
        pltpu.SemaphoreType.DMA,
        pltpu.SemaphoreType.DMA,
        pltpu.SemaphoreType.DMA,
        pltpu.SemaphoreType.DMA,
        pltpu.SemaphoreType.DMA,
    ],
)
def _edge_kernel(src_hbm, dst_hbm, ew_hbm, h2_hbm, out_hbm,
                 src_v, dst_v, ew_v, rows_a, rows_b, acc_sh,
                 sema, semb, ssa, ssb, sema2, semb2, ssa2, ssb2):
    cid = lax.axis_index("c")
    sid = lax.axis_index("s")
    for r in range(8):
        for j in range(H // 16):
            rows_a[r, pl.ds(j * 16, 16)] = jnp.zeros((16,), jnp.float32)
    zsrc = rows_a.at[pl.ds(0, 8)]

    @pl.loop(0, STRIPE // 8)
    def _zero(k):
        pltpu.async_copy(zsrc, acc_sh.at[pl.ds(sid * STRIPE + k * 8, 8)], ssa)

    @pl.loop(0, STRIPE // 8)
    def _zdrain(k):
        pltpu.make_async_copy(
            zsrc, acc_sh.at[pl.ds(sid * STRIPE + k * 8, 8)], ssa).wait()

    plsc.subcore_barrier()
    table = h2_hbm.at[cid]
    HB = EB // 2

    def gather2(bi, rows_v, s1, s2):
        pltpu.async_copy(table.at[src_v.at[bi, pl.ds(0, HB)]],
                         rows_v.at[pl.ds(0, HB)], s1)
        pltpu.async_copy(table.at[src_v.at[bi, pl.ds(HB, HB)]],
                         rows_v.at[pl.ds(HB, HB)], s2)

    def gather2_wait(bi, rows_v, s1, s2):
        pltpu.make_async_copy(table.at[src_v.at[bi, pl.ds(0, HB)]],
                              rows_v.at[pl.ds(0, HB)], s1).wait()
        pltpu.make_async_copy(table.at[src_v.at[bi, pl.ds(HB, HB)]],
                              rows_v.at[pl.ds(HB, HB)], s2).wait()

    def scatter2(bi, rows_v, s1, s2):
        del s2
        pltpu.async_copy(rows_v, acc_sh.at[dst_v.at[bi]], s1, add=True)

    def scatter2_wait(bi, rows_v, s1, s2):
        del s2
        pltpu.make_async_copy(rows_v, acc_sh.at[dst_v.at[bi]], s1).wait()

    def scale(rows_v, bi):
        @pl.loop(0, EB // 16)
        def _grp(g):
            ew16 = ew_v[bi, pl.ds(g * 16, 16)]
            for k in range(16):
                w = ew16[k]
                e = g * 16 + k
                for j in range(H // 16):
                    sl = rows_v[e, pl.ds(j * 16, 16)]
                    rows_v[e, pl.ds(j * 16, 16)] = sl * w

    for p in range(2):
        row0 = sid * RPT + p * RPP
        pltpu.sync_copy(src_hbm.at[pl.ds(row0, RPP)], src_v)
        pltpu.sync_copy(dst_hbm.at[pl.ds(row0, RPP)], dst_v)
        pltpu.sync_copy(ew_hbm.at[pl.ds(row0, RPP)], ew_v)
        gather2(0, rows_a, sema, sema2)
        gather2(1, rows_b, semb, semb2)

        @pl.loop(0, RPP, step=2)
        def _batch(bi):
            gather2_wait(bi, rows_a, sema, sema2)
            scale(rows_a, bi)
            scatter2(bi, rows_a, ssa, ssa2)
            gather2_wait(bi + 1, rows_b, semb, semb2)
            scale(rows_b, bi + 1)
            scatter2(bi + 1, rows_b, ssb, ssb2)
            scatter2_wait(bi, rows_a, ssa, ssa2)

            @pl.when(bi + 2 < RPP)
            def _():
                gather2(bi + 2, rows_a, sema, sema2)

            scatter2_wait(bi + 1, rows_b, ssb, ssb2)

            @pl.when(bi + 3 < RPP)
            def _():
                gather2(bi + 3, rows_b, semb, semb2)

    plsc.subcore_barrier()
    out = out_hbm.at[cid]
    nrb = STRIPE // EB

    pltpu.async_copy(acc_sh.at[pl.ds(sid * STRIPE, EB)], rows_a, sema)

    @pl.loop(0, nrb)
    def _readback(k):
        off = sid * STRIPE + k * EB

        @pl.when(lax.rem(k, 2) == 0)
        def _():
            pltpu.make_async_copy(acc_sh.at[pl.ds(off, EB)], rows_a,
                                  sema).wait()

            @pl.when(k + 1 < nrb)
            def _():
                pltpu.async_copy(
                    acc_sh.at[pl.ds(off + EB, EB)], rows_b, semb)

            pltpu.sync_copy(rows_a, out.at[pl.ds(off, EB)])

        @pl.when(lax.rem(k, 2) == 1)
        def _():
            pltpu.make_async_copy(acc_sh.at[pl.ds(off, EB)], rows_b,
                                  semb).wait()

            @pl.when(k + 1 < nrb)
            def _():
                pltpu.async_copy(
                    acc_sh.at[pl.ds(off + EB, EB)], rows_a, sema)

            pltpu.sync_copy(rows_b, out.at[pl.ds(off, EB)])


def _ln_body(node_ref, h2_ref, acc_ref, dinv_ref, b_ref, g_ref, beta_ref, o_ref):
    h2 = jnp.concatenate([h2_ref[0], h2_ref[1]], axis=1)
    acc = jnp.concatenate([acc_ref[0], acc_ref[1]], axis=1)
    conv = dinv_ref[...] * (acc + h2) + b_ref[...]
    y = node_ref[...] + conv
    mean = jnp.mean(y, axis=-1, keepdims=True)
    var = jnp.mean((y - mean) ** 2, axis=-1, keepdims=True)
    yn = (y - mean) * lax.rsqrt(var + 1e-5) * g_ref[...] + beta_ref[...]
    o_ref[...] = jnp.maximum(yn, 0.0)


def _ln_relu(node, h2s, accs, dinv, b, g, beta, block=200):
    grid = (N // block,)
    return pl.pallas_call(
        _ln_body,
        grid=grid,
        in_specs=[
            pl.BlockSpec((block, D), lambda i: (i, 0)),
            pl.BlockSpec((2, block, H), lambda i: (0, i, 0)),
            pl.BlockSpec((2, block, H), lambda i: (0, i, 0)),
            pl.BlockSpec((block, 1), lambda i: (i, 0)),
            pl.BlockSpec((1, D), lambda i: (0, 0)),
            pl.BlockSpec((1, D), lambda i: (0, 0)),
            pl.BlockSpec((1, D), lambda i: (0, 0)),
        ],
        out_specs=pl.BlockSpec((block, D), lambda i: (i, 0)),
        out_shape=jax.ShapeDtypeStruct((N, D), jnp.float32),
    )(node, h2s, accs, dinv, b, g, beta)


@jax.jit
def kernel(node, edge_index, edge_attr, batch_ptr, W, b, ln_gamma, ln_beta):
    pad = EPAD - E
    src_p = jnp.concatenate([edge_index[0], jnp.zeros((pad,), jnp.int32)])
    dst_p = jnp.concatenate([edge_index[1], jnp.zeros((pad,), jnp.int32)])
    ew2d = jnp.concatenate(
        [edge_attr, jnp.zeros((pad,), jnp.float32)]).reshape(ROWS, EB)
    d0, d1 = _deg_kernel(dst_p.reshape(ROWS, EB), ew2d)
    h2s, dinv = _matmul_prescale(node, W, d0[:N].reshape(N, 1),
                                 d1[:N].reshape(N, 1))
    accs = _edge_kernel(src_p.reshape(ROWS, EB),
                        dst_p.reshape(ROWS, EB), ew2d, h2s)
    return _ln_relu(node, h2s, accs, dinv,
                    b.reshape(1, D), ln_gamma.reshape(1, D),
                    ln_beta.reshape(1, D))


# speedup vs baseline: 1.1078x; 1.1078x over previous
import functools

import jax
import jax.numpy as jnp
from jax import lax
from jax.experimental import pallas as pl
from jax.experimental.pallas import tpu as pltpu
from jax.experimental.pallas import tpu_sc as plsc

N = 10000
E = 160000
D = 256
H = 128
NP = 10240
EB = 128
EPAD = 163840
ROWS = EPAD // EB
RPT = ROWS // 16
RPT_HALF = ROWS // 32
STRIPE = NP // 16

_mesh = plsc.VectorSubcoreMesh(core_axis_name="c", subcore_axis_name="s")


@functools.partial(
    pl.kernel,
    out_type=(
        jax.ShapeDtypeStruct((NP,), jnp.float32),
        jax.ShapeDtypeStruct((NP,), jnp.float32),
    ),
    mesh=_mesh,
    scratch_types=[
        pltpu.VMEM((RPT_HALF, EB), jnp.int32),
        pltpu.VMEM((RPT_HALF, EB), jnp.float32),
        pltpu.VMEM((STRIPE,), jnp.float32),
        pltpu.VMEM_SHARED((NP,), jnp.float32),
        pltpu.SemaphoreType.DMA,
    ],
)
def _deg_kernel(dst_hbm, ew_hbm, d0_hbm, d1_hbm, idx_v, val_v, stripe_v, deg_sh,
                dsem):
    cid = lax.axis_index("c")
    sid = lax.axis_index("s")
    for i in range(STRIPE // 16):
        stripe_v[pl.ds(i * 16, 16)] = jnp.zeros((16,), jnp.float32)
    pltpu.sync_copy(stripe_v, deg_sh.at[pl.ds(sid * STRIPE, STRIPE)])
    plsc.subcore_barrier()
    row0 = cid * (ROWS // 2) + sid * RPT_HALF
    pltpu.sync_copy(dst_hbm.at[pl.ds(row0, RPT_HALF)], idx_v)
    pltpu.sync_copy(ew_hbm.at[pl.ds(row0, RPT_HALF)], val_v)
    for j in range(RPT_HALF):
        pltpu.async_copy(val_v.at[j], deg_sh.at[idx_v.at[j]], dsem, add=True)
    for j in range(RPT_HALF):
        pltpu.make_async_copy(val_v.at[j], deg_sh.at[idx_v.at[j]], dsem).wait()
    plsc.subcore_barrier()
    pltpu.sync_copy(deg_sh.at[pl.ds(sid * STRIPE, STRIPE)], stripe_v)

    @pl.when(cid == 0)
    def _():
        pltpu.sync_copy(stripe_v, d0_hbm.at[pl.ds(sid * STRIPE, STRIPE)])

    @pl.when(cid == 1)
    def _():
        pltpu.sync_copy(stripe_v, d1_hbm.at[pl.ds(sid * STRIPE, STRIPE)])


def _mm_body(node_ref, w_ref, d0_ref, d1_ref, h2_ref, dinv_ref):
    deg = d0_ref[...] + d1_ref[...] + 1.0
    dinv = lax.rsqrt(deg)
    h = jnp.dot(node_ref[...], w_ref[...], preferred_element_type=jnp.float32)
    h2 = h * dinv
    h2_ref[...] = jnp.stack([h2[:, :H], h2[:, H:]])
    dinv_ref[...] = dinv


def _matmul_prescale(node, W, d0, d1, block=1000):
    grid = (N // block,)
    return pl.pallas_call(
        _mm_body,
        grid=grid,
        in_specs=[
            pl.BlockSpec((block, D), lambda i: (i, 0)),
            pl.BlockSpec((D, D), lambda i: (0, 0)),
            pl.BlockSpec((block, 1), lambda i: (i, 0)),
            pl.BlockSpec((block, 1), lambda i: (i, 0)),
        ],
        out_specs=[
            pl.BlockSpec((2, block, H), lambda i: (0, i, 0)),
            pl.BlockSpec((block, 1), lambda i: (i, 0)),
        ],
        out_shape=[
            jax.ShapeDtypeStruct((2, N, H), jnp.float32),
            jax.ShapeDtypeStruct((N, 1), jnp.float32),
        ],
    )(node, W, d0, d1)


RPP = RPT // 2


@functools.partial(
    pl.kernel,
    out_type=jax.ShapeDtypeStruct((2, NP, H), jnp.float32),
    mesh=_mesh,
    scratch_types=[
        pltpu.VMEM((RPP, EB), jnp.int32),
        pltpu.VMEM((RPP, EB), jnp.int32),
        pltpu.VMEM((RPP, EB), jnp.float32),
        pltpu.VMEM((EB, H), jnp.float32),
        pltpu.VMEM((EB, H), jnp.float32),
        pltpu.VMEM_SHARED((NP, H), jnp.float32),
        pltpu.SemaphoreType.DMA,
        pltpu.SemaphoreType.DMA,
        pltpu.SemaphoreType.DMA,
        pltpu.SemaphoreType.DMA,
        pltpu.SemaphoreType.DMA,
        pltpu.SemaphoreType.DMA,
        pltpu.SemaphoreType.DMA,
        pltpu.SemaphoreType.DMA,
    ],
)
def _edge_kernel(src_hbm, dst_hbm, ew_hbm, h2_hbm, out_hbm,
                 src_v, dst_v, ew_v, rows_a, rows_b, acc_sh,
                 sema, semb, ssa, ssb, sema2, semb2, ssa2, ssb2):
    cid = lax.axis_index("c")
    sid = lax.axis_index("s")
    for r in range(8):
        for j in range(H // 16):
            rows_a[r, pl.ds(j * 16, 16)] = jnp.zeros((16,), jnp.float32)
    zsrc = rows_a.at[pl.ds(0, 8)]

    @pl.loop(0, STRIPE // 8)
    def _zero(k):
        pltpu.async_copy(zsrc, acc_sh.at[pl.ds(sid * STRIPE + k * 8, 8)], ssa)

    @pl.loop(0, STRIPE // 8)
    def _zdrain(k):
        pltpu.make_async_copy(
            zsrc, acc_sh.at[pl.ds(sid * STRIPE + k * 8, 8)], ssa).wait()

    plsc.subcore_barrier()
    table = h2_hbm.at[cid]
    HB = EB // 2

    def gather2(bi, rows_v, s1, s2):
        del s2
        pltpu.async_copy(table.at[src_v.at[bi]], rows_v, s1)

    def gather2_wait(bi, rows_v, s1, s2):
        del s2
        pltpu.make_async_copy(table.at[src_v.at[bi]], rows_v, s1).wait()

    def scatter2(bi, rows_v, s1, s2):
        del s2
        pltpu.async_copy(rows_v, acc_sh.at[dst_v.at[bi]], s1, add=True)

    def scatter2_wait(bi, rows_v, s1, s2):
        del s2
        pltpu.make_async_copy(rows_v, acc_sh.at[dst_v.at[bi]], s1).wait()

    def scale(rows_v, bi):
        @pl.loop(0, EB // 16)
        def _grp(g):
            ew16 = ew_v[bi, pl.ds(g * 16, 16)]
            for k in range(16):
                w = ew16[k]
                e = g * 16 + k
                for j in range(H // 16):
                    sl = rows_v[e, pl.ds(j * 16, 16)]
                    rows_v[e, pl.ds(j * 16, 16)] = sl * w

    for p in range(2):
        row0 = sid * RPT + p * RPP
        pltpu.sync_copy(src_hbm.at[pl.ds(row0, RPP)], src_v)
        pltpu.sync_copy(dst_hbm.at[pl.ds(row0, RPP)], dst_v)
        pltpu.sync_copy(ew_hbm.at[pl.ds(row0, RPP)], ew_v)
        gather2(0, rows_a, sema, sema2)
        gather2(1, rows_b, semb, semb2)

        @pl.loop(0, RPP, step=2)
        def _batch(bi):
            gather2_wait(bi, rows_a, sema, sema2)
            scale(rows_a, bi)
            scatter2(bi, rows_a, ssa, ssa2)
            gather2_wait(bi + 1, rows_b, semb, semb2)
            scale(rows_b, bi + 1)
            scatter2(bi + 1, rows_b, ssb, ssb2)
            scatter2_wait(bi, rows_a, ssa, ssa2)

            @pl.when(bi + 2 < RPP)
            def _():
                gather2(bi + 2, rows_a, sema, sema2)

            scatter2_wait(bi + 1, rows_b, ssb, ssb2)

            @pl.when(bi + 3 < RPP)
            def _():
                gather2(bi + 3, rows_b, semb, semb2)

    plsc.subcore_barrier()
    out = out_hbm.at[cid]
    nrb = STRIPE // EB

    pltpu.async_copy(acc_sh.at[pl.ds(sid * STRIPE, EB)], rows_a, sema)

    @pl.loop(0, nrb)
    def _readback(k):
        off = sid * STRIPE + k * EB

        @pl.when(lax.rem(k, 2) == 0)
        def _():
            pltpu.make_async_copy(acc_sh.at[pl.ds(off, EB)], rows_a,
                                  sema).wait()

            @pl.when(k + 1 < nrb)
            def _():
                pltpu.async_copy(
                    acc_sh.at[pl.ds(off + EB, EB)], rows_b, semb)

            pltpu.sync_copy(rows_a, out.at[pl.ds(off, EB)])

        @pl.when(lax.rem(k, 2) == 1)
        def _():
            pltpu.make_async_copy(acc_sh.at[pl.ds(off, EB)], rows_b,
                                  semb).wait()

            @pl.when(k + 1 < nrb)
            def _():
                pltpu.async_copy(
                    acc_sh.at[pl.ds(off + EB, EB)], rows_a, sema)

            pltpu.sync_copy(rows_b, out.at[pl.ds(off, EB)])


def _ln_body(node_ref, h2_ref, acc_ref, dinv_ref, b_ref, g_ref, beta_ref, o_ref):
    h2 = jnp.concatenate([h2_ref[0], h2_ref[1]], axis=1)
    acc = jnp.concatenate([acc_ref[0], acc_ref[1]], axis=1)
    conv = dinv_ref[...] * (acc + h2) + b_ref[...]
    y = node_ref[...] + conv
    mean = jnp.mean(y, axis=-1, keepdims=True)
    var = jnp.mean((y - mean) ** 2, axis=-1, keepdims=True)
    yn = (y - mean) * lax.rsqrt(var + 1e-5) * g_ref[...] + beta_ref[...]
    o_ref[...] = jnp.maximum(yn, 0.0)


def _ln_relu(node, h2s, accs, dinv, b, g, beta, block=1000):
    grid = (N // block,)
    return pl.pallas_call(
        _ln_body,
        grid=grid,
        in_specs=[
            pl.BlockSpec((block, D), lambda i: (i, 0)),
            pl.BlockSpec((2, block, H), lambda i: (0, i, 0)),
            pl.BlockSpec((2, block, H), lambda i: (0, i, 0)),
            pl.BlockSpec((block, 1), lambda i: (i, 0)),
            pl.BlockSpec((1, D), lambda i: (0, 0)),
            pl.BlockSpec((1, D), lambda i: (0, 0)),
            pl.BlockSpec((1, D), lambda i: (0, 0)),
        ],
        out_specs=pl.BlockSpec((block, D), lambda i: (i, 0)),
        out_shape=jax.ShapeDtypeStruct((N, D), jnp.float32),
    )(node, h2s, accs, dinv, b, g, beta)


@jax.jit
def kernel(node, edge_index, edge_attr, batch_ptr, W, b, ln_gamma, ln_beta):
    pad = EPAD - E
    src_p = jnp.concatenate([edge_index[0], jnp.zeros((pad,), jnp.int32)])
    dst_p = jnp.concatenate([edge_index[1], jnp.zeros((pad,), jnp.int32)])
    ew2d = jnp.concatenate(
        [edge_attr, jnp.zeros((pad,), jnp.float32)]).reshape(ROWS, EB)
    d0, d1 = _deg_kernel(dst_p.reshape(ROWS, EB), ew2d)
    h2s, dinv = _matmul_prescale(node, W, d0[:N].reshape(N, 1),
                                 d1[:N].reshape(N, 1))
    accs = _edge_kernel(src_p.reshape(ROWS, EB),
                        dst_p.reshape(ROWS, EB), ew2d, h2s)
    return _ln_relu(node, h2s, accs, dinv,
                    b.reshape(1, D), ln_gamma.reshape(1, D),
                    ln_beta.reshape(1, D))
